# baseline (speedup 1.0000x reference)
import functools

import jax
import jax.numpy as jnp
from jax import lax
from jax.experimental import pallas as pl
from jax.experimental.pallas import tpu as pltpu
from jax.experimental.pallas import tpu_sc as plsc

N_NODES = 5000
OUT = 128
FEAT = 16
E_TOTAL = 320000

L = 16
NC, NS = 2, 16
NW = NC * NS
EPW = E_TOTAL // NW
W = 80
N_WIN = EPW // W
AD = 16
FW = OUT + L


def _proj_body(origin_ref, neighbor_ref, wo_ref, wn_ref,
               wa_ref, z2_ref, asT_ref, adT_ref):
    wa_s = wa_ref[0, 0:OUT]
    wa_d = wa_ref[0, OUT:2 * OUT]
    dn = (((1,), (1,)), ((), ()))
    z2 = lax.dot_general(neighbor_ref[...], wn_ref[...], dn,
                         preferred_element_type=jnp.float32)
    z2_ref[...] = z2
    asrc = z2 @ wa_s
    asT_ref[...] = jnp.broadcast_to(asrc[:, None], (N_NODES, L))
    v_d = lax.dot_general(wa_d, wo_ref[...], (((0,), (0,)), ((), ())),
                          preferred_element_type=jnp.float32)
    adst = origin_ref[...] @ v_d
    adT_ref[...] = jnp.broadcast_to(adst[:, None], (N_NODES, AD))


_tc_proj = pl.pallas_call(
    _proj_body,
    out_shape=(
        jax.ShapeDtypeStruct((N_NODES, OUT), jnp.float32),
        jax.ShapeDtypeStruct((N_NODES, L), jnp.float32),
        jax.ShapeDtypeStruct((N_NODES, AD), jnp.float32),
    ),
)


E_BLK = 8192
E_PAD = 40 * E_BLK


def _af_body(tfidf_ref, wf_ref, wa_ref, afeat_ref):
    wa_f = wa_ref[0, 2 * OUT:3 * OUT]
    w3 = lax.dot_general(wa_f, wf_ref[...], (((0,), (0,)), ((), ())),
                         preferred_element_type=jnp.float32)
    afeat_ref[...] = jnp.sum(tfidf_ref[...] * w3[None, :], axis=1)


_tc_afeat = pl.pallas_call(
    _af_body,
    grid=(E_PAD // E_BLK,),
    in_specs=[
        pl.BlockSpec((E_BLK, FEAT), lambda i: (i, 0)),
        pl.BlockSpec((OUT, FEAT), lambda i: (0, 0)),
        pl.BlockSpec((1, 3 * OUT), lambda i: (0, 0)),
    ],
    out_specs=pl.BlockSpec((E_BLK,), lambda i: (i,)),
    out_shape=jax.ShapeDtypeStruct((E_PAD,), jnp.float32),
)


def _sc_body(src_hbm, dst_hbm, af_hbm, zs_hbm, adT_hbm,
             zn_hbm,
             num_hbm,
             src_w0, dst_w0, af_w0, st_ad0, st_z0,
             src_w1, dst_w1, af_w1, st_ad1, st_z1,
             num_sh,
             sza, szb, saa, sab):
    cid = lax.axis_index("c")
    sid = lax.axis_index("s")
    wid = sid * NC + cid

    @pl.when(sid == 0)
    def _():
        pltpu.sync_copy(zn_hbm, num_sh)

    plsc.subcore_barrier()

    ebase = wid * EPW

    def issue(w, src_w, dst_w, af_w, st_ad, st_z, sz, sa):
        base = ebase + w * W
        pltpu.sync_copy(src_hbm.at[pl.ds(base, W)], src_w)
        pltpu.sync_copy(dst_hbm.at[pl.ds(base, W)], dst_w)
        pltpu.sync_copy(af_hbm.at[pl.ds(base, W)], af_w.at[pl.ds(0, W)])
        pltpu.async_copy(zs_hbm.at[src_w], st_z, sz)
        pltpu.async_copy(adT_hbm.at[dst_w], st_ad, sa)

    def work(src_w, dst_w, af_w, st_ad, st_z, sz, sa):
        pltpu.make_async_copy(zs_hbm.at[src_w], st_z, sz).wait()
        pltpu.make_async_copy(adT_hbm.at[dst_w], st_ad, sa).wait()

        def edge_body(k, c):
            va = st_z[k, pl.ds(OUT, L)]
            vd = st_ad[k]
            vf = af_w[pl.ds(k, L)]
            e = va + vd + vf[0]
            e = jnp.where(e > 0, e, e * jnp.float32(0.01))
            sv = jnp.exp(e)
            for j in range(OUT // L):
                sl = pl.ds(j * L, L)
                st_z[k, sl] = st_z[k, sl] * sv
            st_z[k, pl.ds(OUT, L)] = sv
            return c

        lax.fori_loop(0, W, edge_body, 0)
        pltpu.sync_copy(st_z, num_sh.at[dst_w], add=True)

    issue(0, src_w0, dst_w0, af_w0, st_ad0, st_z0, sza, saa)

    def window_body(w, carry):
        @pl.when(w % 2 == 0)
        def _():
            @pl.when(w + 1 < N_WIN)
            def _():
                issue(w + 1, src_w1, dst_w1, af_w1, st_ad1, st_z1, szb, sab)
            work(src_w0, dst_w0, af_w0, st_ad0, st_z0, sza, saa)

        @pl.when(w % 2 == 1)
        def _():
            @pl.when(w + 1 < N_WIN)
            def _():
                issue(w + 1, src_w0, dst_w0, af_w0, st_ad0, st_z0, sza, saa)
            work(src_w1, dst_w1, af_w1, st_ad1, st_z1, szb, sab)

        return carry

    lax.fori_loop(0, N_WIN, window_body, 0)

    plsc.subcore_barrier()

    @pl.when(sid == 0)
    def _():
        pltpu.sync_copy(num_sh, num_hbm.at[cid])


_sc_edge = functools.partial(
    pl.kernel,
    out_type=(
        jax.ShapeDtypeStruct((NC, N_NODES, FW), jnp.float32),
    ),
    mesh=plsc.VectorSubcoreMesh(core_axis_name="c", subcore_axis_name="s"),
    compiler_params=pltpu.CompilerParams(use_tc_tiling_on_sc=False),
    scratch_types=[
        pltpu.VMEM((W,), jnp.int32),
        pltpu.VMEM((W,), jnp.int32),
        pltpu.VMEM((W + L,), jnp.float32),
        pltpu.VMEM((W, AD), jnp.float32),
        pltpu.VMEM((W, FW), jnp.float32),
        pltpu.VMEM((W,), jnp.int32),
        pltpu.VMEM((W,), jnp.int32),
        pltpu.VMEM((W + L,), jnp.float32),
        pltpu.VMEM((W, AD), jnp.float32),
        pltpu.VMEM((W, FW), jnp.float32),
        pltpu.VMEM_SHARED((N_NODES, FW), jnp.float32),
        pltpu.SemaphoreType.DMA,
        pltpu.SemaphoreType.DMA,
        pltpu.SemaphoreType.DMA,
        pltpu.SemaphoreType.DMA,
    ],
)(_sc_body)


def _combine_body(num_ref, out_ref):
    n = num_ref[0, :, 0:OUT] + num_ref[1, :, 0:OUT]
    d = num_ref[0, :, OUT] + num_ref[1, :, OUT]
    d = jnp.where(d > 0, d, jnp.float32(1.0))
    out_ref[...] = n / d[:, None]


_tc_combine = pl.pallas_call(
    _combine_body,
    out_shape=jax.ShapeDtypeStruct((N_NODES, OUT), jnp.float32),
)


def kernel(origin, neighbor, edge_index, tfidfembed, W_o, W_n, W_feat, W_attn):
    src = edge_index[0]
    dst = edge_index[1]
    z2, asT, adT = _tc_proj(origin, neighbor, W_o, W_n, W_attn)
    zs = jnp.concatenate([z2, asT], axis=1)
    tfp = jnp.pad(tfidfembed, ((0, E_PAD - E_TOTAL), (0, 0)))
    afeat = _tc_afeat(tfp, W_feat, W_attn)[:E_TOTAL]
    zn = jnp.zeros((N_NODES, FW), jnp.float32)
    (num,) = _sc_edge(src, dst, afeat, zs, adT, zn)
    return _tc_combine(num)


# speedup vs baseline: 1.2031x; 1.2031x over previous
import functools

import jax
import jax.numpy as jnp
from jax import lax
from jax.experimental import pallas as pl
from jax.experimental.pallas import tpu as pltpu
from jax.experimental.pallas import tpu_sc as plsc

N_NODES = 5000
OUT = 128
FEAT = 16
E_TOTAL = 320000

L = 16
NC, NS = 2, 16
NW = NC * NS
EPW = E_TOTAL // NW
W = 80
N_WIN = EPW // W
AD = 16
FW = OUT + L


def _proj_body(origin_ref, neighbor_ref, wo_ref, wn_ref,
               wa_ref, z2_ref, asT_ref, adT_ref):
    wa_s = wa_ref[0, 0:OUT]
    wa_d = wa_ref[0, OUT:2 * OUT]
    dn = (((1,), (1,)), ((), ()))
    z2 = lax.dot_general(neighbor_ref[...], wn_ref[...], dn,
                         preferred_element_type=jnp.float32)
    z2_ref[...] = z2
    asrc = z2 @ wa_s
    asT_ref[...] = jnp.broadcast_to(asrc[:, None], (N_NODES, L))
    v_d = lax.dot_general(wa_d, wo_ref[...], (((0,), (0,)), ((), ())),
                          preferred_element_type=jnp.float32)
    adst = origin_ref[...] @ v_d
    adT_ref[...] = jnp.broadcast_to(adst[:, None], (N_NODES, AD))


_tc_proj = pl.pallas_call(
    _proj_body,
    out_shape=(
        jax.ShapeDtypeStruct((N_NODES, OUT), jnp.float32),
        jax.ShapeDtypeStruct((N_NODES, L), jnp.float32),
        jax.ShapeDtypeStruct((N_NODES, AD), jnp.float32),
    ),
)


E_BLK = 8192
E_PAD = 40 * E_BLK


def _af_body(tfidf_ref, wf_ref, wa_ref, afeat_ref):
    wa_f = wa_ref[0, 2 * OUT:3 * OUT]
    w3 = lax.dot_general(wa_f, wf_ref[...], (((0,), (0,)), ((), ())),
                         preferred_element_type=jnp.float32)
    afeat_ref[...] = jnp.sum(tfidf_ref[...] * w3[None, :], axis=1)


_tc_afeat = pl.pallas_call(
    _af_body,
    grid=(E_PAD // E_BLK,),
    in_specs=[
        pl.BlockSpec((E_BLK, FEAT), lambda i: (i, 0)),
        pl.BlockSpec((OUT, FEAT), lambda i: (0, 0)),
        pl.BlockSpec((1, 3 * OUT), lambda i: (0, 0)),
    ],
    out_specs=pl.BlockSpec((E_BLK,), lambda i: (i,)),
    out_shape=jax.ShapeDtypeStruct((E_PAD,), jnp.float32),
)


def _sc_body(src_hbm, dst_hbm, af_hbm, zs_hbm, adT_hbm,
             zn_hbm,
             num_hbm,
             src_w0, dst_w0, af_w0, st_ad0, st_z0,
             src_w1, dst_w1, af_w1, st_ad1, st_z1,
             num_sh,
             sza, szb, saa, sab):
    cid = lax.axis_index("c")
    sid = lax.axis_index("s")
    wid = sid * NC + cid

    @pl.when(sid == 0)
    def _():
        pltpu.sync_copy(zn_hbm, num_sh)

    plsc.subcore_barrier()

    ebase = wid * EPW

    def issue(w, src_w, dst_w, af_w, st_ad, st_z, sz, sa):
        base = ebase + w * W
        pltpu.sync_copy(src_hbm.at[pl.ds(base, W)], src_w)
        pltpu.sync_copy(dst_hbm.at[pl.ds(base, W)], dst_w)
        pltpu.sync_copy(af_hbm.at[pl.ds(base, W)], af_w.at[pl.ds(0, W)])
        pltpu.async_copy(zs_hbm.at[src_w], st_z, sz)
        pltpu.async_copy(adT_hbm.at[dst_w], st_ad, sa)

    def work(src_w, dst_w, af_w, st_ad, st_z, sz, sa):
        pltpu.make_async_copy(zs_hbm.at[src_w], st_z, sz).wait()
        pltpu.make_async_copy(adT_hbm.at[dst_w], st_ad, sa).wait()

        @plsc.parallel_loop(0, W, unroll=4)
        def edge_body(k):
            va = st_z[k, pl.ds(OUT, L)]
            vd = st_ad[k]
            vf = af_w[pl.ds(k, L)]
            e = va + vd + vf[0]
            e = jnp.where(e > 0, e, e * jnp.float32(0.01))
            sv = jnp.exp(e)
            for j in range(OUT // L):
                sl = pl.ds(j * L, L)
                st_z[k, sl] = st_z[k, sl] * sv
            st_z[k, pl.ds(OUT, L)] = sv
        pltpu.sync_copy(st_z, num_sh.at[dst_w], add=True)

    issue(0, src_w0, dst_w0, af_w0, st_ad0, st_z0, sza, saa)

    def window_body(w, carry):
        @pl.when(w % 2 == 0)
        def _():
            @pl.when(w + 1 < N_WIN)
            def _():
                issue(w + 1, src_w1, dst_w1, af_w1, st_ad1, st_z1, szb, sab)
            work(src_w0, dst_w0, af_w0, st_ad0, st_z0, sza, saa)

        @pl.when(w % 2 == 1)
        def _():
            @pl.when(w + 1 < N_WIN)
            def _():
                issue(w + 1, src_w0, dst_w0, af_w0, st_ad0, st_z0, sza, saa)
            work(src_w1, dst_w1, af_w1, st_ad1, st_z1, szb, sab)

        return carry

    lax.fori_loop(0, N_WIN, window_body, 0)

    plsc.subcore_barrier()

    @pl.when(sid == 0)
    def _():
        pltpu.sync_copy(num_sh, num_hbm.at[cid])


_sc_edge = functools.partial(
    pl.kernel,
    out_type=(
        jax.ShapeDtypeStruct((NC, N_NODES, FW), jnp.float32),
    ),
    mesh=plsc.VectorSubcoreMesh(core_axis_name="c", subcore_axis_name="s"),
    compiler_params=pltpu.CompilerParams(use_tc_tiling_on_sc=False),
    scratch_types=[
        pltpu.VMEM((W,), jnp.int32),
        pltpu.VMEM((W,), jnp.int32),
        pltpu.VMEM((W + L,), jnp.float32),
        pltpu.VMEM((W, AD), jnp.float32),
        pltpu.VMEM((W, FW), jnp.float32),
        pltpu.VMEM((W,), jnp.int32),
        pltpu.VMEM((W,), jnp.int32),
        pltpu.VMEM((W + L,), jnp.float32),
        pltpu.VMEM((W, AD), jnp.float32),
        pltpu.VMEM((W, FW), jnp.float32),
        pltpu.VMEM_SHARED((N_NODES, FW), jnp.float32),
        pltpu.SemaphoreType.DMA,
        pltpu.SemaphoreType.DMA,
        pltpu.SemaphoreType.DMA,
        pltpu.SemaphoreType.DMA,
    ],
)(_sc_body)


def _combine_body(num_ref, out_ref):
    n = num_ref[0, :, 0:OUT] + num_ref[1, :, 0:OUT]
    d = num_ref[0, :, OUT] + num_ref[1, :, OUT]
    d = jnp.where(d > 0, d, jnp.float32(1.0))
    out_ref[...] = n / d[:, None]


_tc_combine = pl.pallas_call(
    _combine_body,
    out_shape=jax.ShapeDtypeStruct((N_NODES, OUT), jnp.float32),
)


def kernel(origin, neighbor, edge_index, tfidfembed, W_o, W_n, W_feat, W_attn):
    src = edge_index[0]
    dst = edge_index[1]
    z2, asT, adT = _tc_proj(origin, neighbor, W_o, W_n, W_attn)
    zs = jnp.concatenate([z2, asT], axis=1)
    tfp = jnp.pad(tfidfembed, ((0, E_PAD - E_TOTAL), (0, 0)))
    afeat = _tc_afeat(tfp, W_feat, W_attn)[:E_TOTAL]
    zn = jnp.zeros((N_NODES, FW), jnp.float32)
    (num,) = _sc_edge(src, dst, afeat, zs, adT, zn)
    return _tc_combine(num)
